# baseline (speedup 1.0000x reference)
import jax
import jax.numpy as jnp
from jax import lax
from jax.experimental import pallas as pl
from jax.experimental.pallas import tpu as pltpu
from jax.experimental.pallas import tpu_sc as plsc

NC = 2
NS = 16
NW = NC * NS

UNIT = 64
K_SUB = 20
CHUNK = UNIT * K_SUB
VCH = 8
LANES = 16
ACCR = 632


def _xw_body(x_ref, w_ref, b_ref, o_ref):
    o_ref[...] = (
        jnp.dot(x_ref[...], w_ref[...], preferred_element_type=jnp.float32)
        + b_ref[...]
    )


def _ew_body(ea_ref, w_ref, o_ref):
    o_ref[...] = jnp.dot(ea_ref[...].astype(jnp.bfloat16),
                         w_ref[...].astype(jnp.bfloat16),
                         preferred_element_type=jnp.float32
                         ).astype(jnp.bfloat16)


def _head_body(x_ref, p_ref, av_ref, wu1_ref, wu2_ref, bu_ref, wa_ref,
               ba_ref, o_ref):
    agg = p_ref[0] + p_ref[1]
    h = jnp.maximum(
        jnp.dot(x_ref[...], wu1_ref[...], preferred_element_type=jnp.float32)
        + jnp.dot(agg, wu2_ref[...], preferred_element_type=jnp.float32)
        + bu_ref[...],
        0.0,
    )
    logit = jnp.dot(h, wa_ref[...], preferred_element_type=jnp.float32) + ba_ref[...]
    logit = jnp.where(jnp.isnan(logit), 0.0, logit)
    logit = jnp.clip(logit, -1000000.0, 1000000.0)
    logit = jnp.where(av_ref[...] == 0, -10000000000.0, logit)
    m = jnp.max(logit, axis=-1, keepdims=True)
    e = jnp.exp(logit - m)
    pi = e / jnp.sum(e, axis=-1, keepdims=True)
    pi = jnp.where(jnp.isnan(pi), 1e-10, pi)
    o_ref[...] = pi / jnp.sum(pi, axis=-1, keepdims=True)


def _sc_body(xw, ew, src1, dst1, out, acc, ibs, ibd, rows3, ewp,
             gsem, esem, ssem, isem):
    H = xw.shape[1]
    E = ew.shape[0] * 2 // H
    total_chunks = E // CHUNK
    nsteps = VCH * K_SUB

    c = lax.axis_index("c")
    s = lax.axis_index("s")
    wid = s * NC + c

    def zrow(r, carry):
        for v in range(H // LANES):
            rows3[0, r, pl.ds(v * LANES, LANES)] = jnp.zeros((LANES,),
                                                             jnp.float32)
        return carry

    lax.fori_loop(0, UNIT, zrow, 0)
    for t in range(ACCR // UNIT):
        pltpu.sync_copy(rows3.at[0], acc.at[pl.ds(s * ACCR + t * UNIT, UNIT)])
    rem = ACCR - (ACCR // UNIT) * UNIT
    if rem:
        pltpu.sync_copy(
            rows3.at[0, pl.ds(0, rem)],
            acc.at[pl.ds(s * ACCR + ACCR - rem, rem)])
    plsc.subcore_barrier()

    def chunk_of(t):
        j = t // K_SUB
        u = t - j * K_SUB
        ci = wid + j * NW
        real = ci < total_chunks
        return j, u, jnp.minimum(ci, total_chunks - 1), real

    def idx_descs(j):
        ci = jnp.minimum(wid + j * NW, total_chunks - 1)
        p = j % 2
        base = ci * CHUNK
        d = [pltpu.make_async_copy(src1.at[pl.ds(base, CHUNK)],
                                   ibs.at[p], isem.at[p])]
        for tt in range(K_SUB):
            d.append(pltpu.make_async_copy(
                dst1.at[pl.ds(base + tt * UNIT, UNIT)],
                ibd.at[p, tt], isem.at[p]))
        return d

    def ge_descs(t):
        j, u, ci, _ = chunk_of(t)
        g = pltpu.make_async_copy(
            xw.at[ibs.at[j % 2, pl.ds(u * UNIT, UNIT)]],
            rows3.at[t % 3], gsem.at[t % 3])
        e = pltpu.make_async_copy(
            ew.at[pl.ds((ci * CHUNK + u * UNIT) * (H // 2), UNIT * H // 2)],
            ewp.at[pl.ds((t % 2) * (UNIT * H // 2), UNIT * H // 2)],
            esem.at[t % 2])
        return g, e

    def s_desc(t):
        j, u, _, real = chunk_of(t)
        d = pltpu.make_async_copy(rows3.at[t % 3],
                                  acc.at[ibd.at[j % 2, u]], ssem.at[t % 3])
        return d, real

    for d in idx_descs(0):
        d.start()
    for d in idx_descs(0):
        d.wait()
    g0, e0 = ge_descs(0)
    g0.start()
    e0.start()

    def step(t, carry):
        j, u, ci, real = chunk_of(t)

        @pl.when(t >= 2)
        def _wait_prev_scatter():
            d, r = s_desc(t - 2)

            @pl.when(r)
            def _w():
                d.wait()

        @pl.when(t + 1 < nsteps)
        def _issue_next():
            g, e = ge_descs(t + 1)
            g.start()
            e.start()

        g, e = ge_descs(t)
        g.wait()
        e.wait()

        @pl.when((u == 1) & (j + 1 < VCH))
        def _prefetch_idx():
            for d in idx_descs(j + 1):
                d.start()

        @pl.when((u == K_SUB - 2) & (j + 1 < VCH))
        def _drain_idx():
            for d in idx_descs(j + 1):
                d.wait()

        pg = t % 3
        pe = t % 2

        @plsc.parallel_loop(0, UNIT)
        def _row(r):
            for q in range(H // 32):
                off = pe * (UNIT * H // 2) + r * (H // 2) + q * 16
                iv = ewp[pl.ds(off, 16)]
                a = jax.lax.bitcast_convert_type(iv << 16, jnp.float32)
                b = jax.lax.bitcast_convert_type(iv & jnp.int32(-65536),
                                                 jnp.float32)
                sla = pl.ds(q * 16, 16)
                slb = pl.ds(H // 2 + q * 16, 16)
                rows3[pg, r, sla] = jnp.maximum(rows3[pg, r, sla] + a, 0.0)
                rows3[pg, r, slb] = jnp.maximum(rows3[pg, r, slb] + b, 0.0)

        sd, _ = s_desc(t)

        @pl.when(real)
        def _issue_scatter():
            sd.start(add=True)

        return carry

    lax.fori_loop(0, nsteps, step, 0)

    for t in (nsteps - 2, nsteps - 1):
        d, r = s_desc(t)

        @pl.when(r)
        def _wait_tail(d=d):
            d.wait()

    plsc.subcore_barrier()

    for t in range(ACCR // UNIT):
        pltpu.sync_copy(acc.at[pl.ds(s * ACCR + t * UNIT, UNIT)], rows3.at[0])
        pltpu.sync_copy(rows3.at[0],
                        out.at[c, pl.ds(s * ACCR + t * UNIT, UNIT)])
    if ACCR - (ACCR // UNIT) * UNIT:
        rem = ACCR - (ACCR // UNIT) * UNIT
        pltpu.sync_copy(acc.at[pl.ds(s * ACCR + ACCR - rem, rem)],
                        rows3.at[0, pl.ds(0, rem)])
        pltpu.sync_copy(rows3.at[0, pl.ds(0, rem)],
                        out.at[c, pl.ds(s * ACCR + ACCR - rem, rem)])


def _sc_segment_mlp(xw, ew, src1, dst1):
    N, H = xw.shape
    npad = NS * ACCR
    return pl.kernel(
        _sc_body,
        out_type=jax.ShapeDtypeStruct((NC, npad, H), jnp.float32),
        mesh=plsc.VectorSubcoreMesh(core_axis_name="c", subcore_axis_name="s"),
        scratch_types=[
            pltpu.VMEM_SHARED((npad, H), jnp.float32),
            pltpu.VMEM((2, CHUNK), jnp.int32),
            pltpu.VMEM((2, K_SUB, UNIT), jnp.int32),
            pltpu.VMEM((3, UNIT, H), jnp.float32),
            pltpu.VMEM((UNIT * 128,), jnp.int32),
            pltpu.SemaphoreType.DMA((3,)),
            pltpu.SemaphoreType.DMA((2,)),
            pltpu.SemaphoreType.DMA((3,)),
            pltpu.SemaphoreType.DMA((2,)),
        ],
    )(xw, ew, src1, dst1)


def kernel(obs, avail_actions, edge_index, edge_attr,
           W_msg, b_msg, W_upd, b_upd, W_act, b_act):
    B, A, F = obs.shape
    N = B * A
    E = edge_index.shape[1]
    H = W_msg.shape[1]
    NA = W_act.shape[1]
    DE = W_msg.shape[0] - F
    assert E % CHUNK == 0 and NW * VCH * CHUNK >= E and H % LANES == 0
    assert NS * ACCR >= N

    x = obs.reshape(N, F)
    W1 = W_msg[:F]
    W2 = W_msg[F:]

    NB = 10
    R = N // NB
    xw = pl.pallas_call(
        _xw_body,
        grid=(NB,),
        in_specs=[
            pl.BlockSpec((R, F), lambda i: (i, 0)),
            pl.BlockSpec((F, H), lambda i: (0, 0)),
            pl.BlockSpec((1, H), lambda i: (0, 0)),
        ],
        out_specs=pl.BlockSpec((R, H), lambda i: (i, 0)),
        out_shape=jax.ShapeDtypeStruct((N, H), jnp.float32),
    )(x, W1, b_msg.reshape(1, H))

    perm = jnp.stack([jnp.arange(H // 2), jnp.arange(H // 2) + H // 2],
                     axis=1).reshape(H)
    EB = 2000
    ew = pl.pallas_call(
        _ew_body,
        grid=(E // EB,),
        in_specs=[
            pl.BlockSpec((EB, DE), lambda i: (i, 0)),
            pl.BlockSpec((DE, H), lambda i: (0, 0)),
        ],
        out_specs=pl.BlockSpec((EB, H), lambda i: (i, 0)),
        out_shape=jax.ShapeDtypeStruct((E, H), jnp.bfloat16),
    )(edge_attr, W2[:, perm])

    ew1 = jax.lax.bitcast_convert_type(
        ew.reshape(E * H // 2, 2), jnp.int32)
    parts = _sc_segment_mlp(xw, ew1, edge_index[0], edge_index[1])

    pi = pl.pallas_call(
        _head_body,
        grid=(NB,),
        in_specs=[
            pl.BlockSpec((R, F), lambda i: (i, 0)),
            pl.BlockSpec((NC, R, H), lambda i: (0, i, 0)),
            pl.BlockSpec((R, NA), lambda i: (i, 0)),
            pl.BlockSpec((F, H), lambda i: (0, 0)),
            pl.BlockSpec((H, H), lambda i: (0, 0)),
            pl.BlockSpec((1, H), lambda i: (0, 0)),
            pl.BlockSpec((H, NA), lambda i: (0, 0)),
            pl.BlockSpec((1, NA), lambda i: (0, 0)),
        ],
        out_specs=pl.BlockSpec((R, NA), lambda i: (i, 0)),
        out_shape=jax.ShapeDtypeStruct((N, NA), jnp.float32),
    )(x, parts, avail_actions.reshape(N, NA), W_upd[:F], W_upd[F:],
      b_upd.reshape(1, H), W_act, b_act.reshape(1, NA))

    return pi.reshape(B, A, NA)


# speedup vs baseline: 32.9599x; 32.9599x over previous
import jax
import jax.numpy as jnp
from jax import lax
from jax.experimental import pallas as pl
from jax.experimental.pallas import tpu as pltpu
from jax.experimental.pallas import tpu_sc as plsc

NC = 2
NS = 16
NW = NC * NS

UNIT = 64
K_SUB = 20
CHUNK = UNIT * K_SUB
VCH = 8
LANES = 16
ACCR = 632


def _xw_body(x_ref, w_ref, b_ref, o_ref):
    o_ref[...] = (
        jnp.dot(x_ref[...], w_ref[...], preferred_element_type=jnp.float32)
        + b_ref[...]
    )


def _ew_body(ea_ref, w_ref, o_ref):
    o_ref[...] = jnp.dot(ea_ref[...].astype(jnp.bfloat16),
                         w_ref[...].astype(jnp.bfloat16),
                         preferred_element_type=jnp.float32)


def _head_body(x_ref, p_ref, av_ref, wu1_ref, wu2_ref, bu_ref, wa_ref,
               ba_ref, o_ref):
    agg = p_ref[0] + p_ref[1]
    h = jnp.maximum(
        jnp.dot(x_ref[...], wu1_ref[...], preferred_element_type=jnp.float32)
        + jnp.dot(agg, wu2_ref[...], preferred_element_type=jnp.float32)
        + bu_ref[...],
        0.0,
    )
    logit = jnp.dot(h, wa_ref[...], preferred_element_type=jnp.float32) + ba_ref[...]
    logit = jnp.where(jnp.isnan(logit), 0.0, logit)
    logit = jnp.clip(logit, -1000000.0, 1000000.0)
    logit = jnp.where(av_ref[...] == 0, -10000000000.0, logit)
    m = jnp.max(logit, axis=-1, keepdims=True)
    e = jnp.exp(logit - m)
    pi = e / jnp.sum(e, axis=-1, keepdims=True)
    pi = jnp.where(jnp.isnan(pi), 1e-10, pi)
    o_ref[...] = pi / jnp.sum(pi, axis=-1, keepdims=True)


def _sc_body(xw, ew, src1, dst1, out, acc, ibs, ibd, rows3, ewp,
             gsem, esem, ssem, isem):
    E = ew.shape[0] * 8
    H = xw.shape[1]
    total_chunks = E // CHUNK
    nsteps = VCH * K_SUB

    c = lax.axis_index("c")
    s = lax.axis_index("s")
    wid = s * NC + c

    def zrow(r, carry):
        for v in range(H // LANES):
            rows3[0, r, pl.ds(v * LANES, LANES)] = jnp.zeros((LANES,),
                                                             jnp.float32)
        return carry

    lax.fori_loop(0, UNIT, zrow, 0)
    for t in range(ACCR // UNIT):
        pltpu.sync_copy(rows3.at[0], acc.at[pl.ds(s * ACCR + t * UNIT, UNIT)])
    rem = ACCR - (ACCR // UNIT) * UNIT
    if rem:
        pltpu.sync_copy(
            rows3.at[0, pl.ds(0, rem)],
            acc.at[pl.ds(s * ACCR + ACCR - rem, rem)])
    plsc.subcore_barrier()

    def chunk_of(t):
        j = t // K_SUB
        u = t - j * K_SUB
        ci = wid + j * NW
        real = ci < total_chunks
        return j, u, jnp.minimum(ci, total_chunks - 1), real

    def idx_descs(j):
        ci = jnp.minimum(wid + j * NW, total_chunks - 1)
        p = j % 2
        base = ci * CHUNK
        d = [pltpu.make_async_copy(src1.at[pl.ds(base, CHUNK)],
                                   ibs.at[p], isem.at[p])]
        for tt in range(K_SUB):
            d.append(pltpu.make_async_copy(
                dst1.at[pl.ds(base + tt * UNIT, UNIT)],
                ibd.at[p, tt], isem.at[p]))
        return d

    def ge_descs(t):
        j, u, ci, _ = chunk_of(t)
        g = pltpu.make_async_copy(
            xw.at[ibs.at[j % 2, pl.ds(u * UNIT, UNIT)]],
            rows3.at[t % 3], gsem.at[t % 3])
        e = pltpu.make_async_copy(
            ew.at[pl.ds(ci * (CHUNK // 8) + u * (UNIT // 8), UNIT // 8)],
            ewp.at[t % 2], esem.at[t % 2])
        return g, e

    def s_desc(t):
        j, u, _, real = chunk_of(t)
        d = pltpu.make_async_copy(rows3.at[t % 3],
                                  acc.at[ibd.at[j % 2, u]], ssem.at[t % 3])
        return d, real

    for d in idx_descs(0):
        d.start()
    for d in idx_descs(0):
        d.wait()
    g0, e0 = ge_descs(0)
    g0.start()
    e0.start()

    def step(t, carry):
        j, u, ci, real = chunk_of(t)

        @pl.when(t >= 2)
        def _wait_prev_scatter():
            d, r = s_desc(t - 2)

            @pl.when(r)
            def _w():
                d.wait()

        @pl.when(t + 1 < nsteps)
        def _issue_next():
            g, e = ge_descs(t + 1)
            g.start()
            e.start()

        g, e = ge_descs(t)
        g.wait()
        e.wait()

        @pl.when((u == 1) & (j + 1 < VCH))
        def _prefetch_idx():
            for d in idx_descs(j + 1):
                d.start()

        @pl.when((u == K_SUB - 2) & (j + 1 < VCH))
        def _drain_idx():
            for d in idx_descs(j + 1):
                d.wait()

        pg = t % 3
        pe = t % 2

        @plsc.parallel_loop(0, UNIT // 8)
        def _row(rr):
            for kk in range(8):
                for v in range(H // LANES):
                    sl = pl.ds(v * LANES, LANES)
                    esl = pl.ds(kk * H + v * LANES, LANES)
                    rows3[pg, rr * 8 + kk, sl] = jnp.maximum(
                        rows3[pg, rr * 8 + kk, sl] + ewp[pe, rr, esl], 0.0)

        sd, _ = s_desc(t)

        @pl.when(real)
        def _issue_scatter():
            sd.start(add=True)

        return carry

    lax.fori_loop(0, nsteps, step, 0)

    for t in (nsteps - 2, nsteps - 1):
        d, r = s_desc(t)

        @pl.when(r)
        def _wait_tail(d=d):
            d.wait()

    plsc.subcore_barrier()

    for t in range(ACCR // UNIT):
        pltpu.sync_copy(acc.at[pl.ds(s * ACCR + t * UNIT, UNIT)], rows3.at[0])
        pltpu.sync_copy(rows3.at[0],
                        out.at[c, pl.ds(s * ACCR + t * UNIT, UNIT)])
    if ACCR - (ACCR // UNIT) * UNIT:
        rem = ACCR - (ACCR // UNIT) * UNIT
        pltpu.sync_copy(acc.at[pl.ds(s * ACCR + ACCR - rem, rem)],
                        rows3.at[0, pl.ds(0, rem)])
        pltpu.sync_copy(rows3.at[0, pl.ds(0, rem)],
                        out.at[c, pl.ds(s * ACCR + ACCR - rem, rem)])


def _sc_segment_mlp(xw, ew, src1, dst1):
    N, H = xw.shape
    npad = NS * ACCR
    return pl.kernel(
        _sc_body,
        out_type=jax.ShapeDtypeStruct((NC, npad, H), jnp.float32),
        mesh=plsc.VectorSubcoreMesh(core_axis_name="c", subcore_axis_name="s"),
        scratch_types=[
            pltpu.VMEM_SHARED((npad, H), jnp.float32),
            pltpu.VMEM((2, CHUNK), jnp.int32),
            pltpu.VMEM((2, K_SUB, UNIT), jnp.int32),
            pltpu.VMEM((3, UNIT, H), jnp.float32),
            pltpu.VMEM((2, UNIT // 8, 8 * H), jnp.float32),
            pltpu.SemaphoreType.DMA((3,)),
            pltpu.SemaphoreType.DMA((2,)),
            pltpu.SemaphoreType.DMA((3,)),
            pltpu.SemaphoreType.DMA((2,)),
        ],
    )(xw, ew, src1, dst1)


def kernel(obs, avail_actions, edge_index, edge_attr,
           W_msg, b_msg, W_upd, b_upd, W_act, b_act):
    B, A, F = obs.shape
    N = B * A
    E = edge_index.shape[1]
    H = W_msg.shape[1]
    NA = W_act.shape[1]
    DE = W_msg.shape[0] - F
    assert E % CHUNK == 0 and NW * VCH * CHUNK >= E and H % LANES == 0
    assert NS * ACCR >= N

    x = obs.reshape(N, F)
    W1 = W_msg[:F]
    W2 = W_msg[F:]

    NB = 10
    R = N // NB
    xw = pl.pallas_call(
        _xw_body,
        grid=(NB,),
        in_specs=[
            pl.BlockSpec((R, F), lambda i: (i, 0)),
            pl.BlockSpec((F, H), lambda i: (0, 0)),
            pl.BlockSpec((1, H), lambda i: (0, 0)),
        ],
        out_specs=pl.BlockSpec((R, H), lambda i: (i, 0)),
        out_shape=jax.ShapeDtypeStruct((N, H), jnp.float32),
    )(x, W1, b_msg.reshape(1, H))

    E8 = E // 8
    ea8 = edge_attr.reshape(E8, 8 * DE)
    w2blk = jnp.kron(jnp.eye(8, dtype=W2.dtype), W2)
    EB8 = 1000
    ew = pl.pallas_call(
        _ew_body,
        grid=(E8 // EB8,),
        in_specs=[
            pl.BlockSpec((EB8, 8 * DE), lambda i: (i, 0)),
            pl.BlockSpec((8 * DE, 8 * H), lambda i: (0, 0)),
        ],
        out_specs=pl.BlockSpec((EB8, 8 * H), lambda i: (i, 0)),
        out_shape=jax.ShapeDtypeStruct((E8, 8 * H), jnp.float32),
    )(ea8, w2blk)

    parts = _sc_segment_mlp(xw, ew, edge_index[0], edge_index[1])

    pi = pl.pallas_call(
        _head_body,
        grid=(NB,),
        in_specs=[
            pl.BlockSpec((R, F), lambda i: (i, 0)),
            pl.BlockSpec((NC, R, H), lambda i: (0, i, 0)),
            pl.BlockSpec((R, NA), lambda i: (i, 0)),
            pl.BlockSpec((F, H), lambda i: (0, 0)),
            pl.BlockSpec((H, H), lambda i: (0, 0)),
            pl.BlockSpec((1, H), lambda i: (0, 0)),
            pl.BlockSpec((H, NA), lambda i: (0, 0)),
            pl.BlockSpec((1, NA), lambda i: (0, 0)),
        ],
        out_specs=pl.BlockSpec((R, NA), lambda i: (i, 0)),
        out_shape=jax.ShapeDtypeStruct((N, NA), jnp.float32),
    )(x, parts, avail_actions.reshape(N, NA), W_upd[:F], W_upd[F:],
      b_upd.reshape(1, H), W_act, b_act.reshape(1, NA))

    return pi.reshape(B, A, NA)
